# Initial kernel scaffold; baseline (speedup 1.0000x reference)
import jax
import jax.numpy as jnp
from jax.experimental import pallas as pl


def kernel(vid, dists, inds):
    raise NotImplementedError("write your pallas kernel here")


# speedup vs baseline: 82.1528x; 82.1528x over previous
import functools

import jax
import jax.numpy as jnp
import numpy as np
from jax import lax
from jax.experimental import pallas as pl
from jax.experimental.pallas import tpu as pltpu
from jax.experimental.pallas import tpu_sc as plsc

PS = 5
K_TOP = 10

NUM_CORES = 2
NUM_SUBCORES = 16
NUM_TILES = NUM_CORES * NUM_SUBCORES
LANES = 16


def _make_sc_call(HD, T, C, H, W, Q):
    assert C == LANES
    n_rows = HD * T * H * W
    q_per_tile = Q // NUM_TILES
    meta_row = 3 * LANES
    n_idx = 256
    pp = PS * PS

    mesh = plsc.VectorSubcoreMesh(
        core_axis_name="c", subcore_axis_name="s",
        num_cores=NUM_CORES, num_subcores=NUM_SUBCORES)

    @functools.partial(
        pl.kernel,
        out_type=jax.ShapeDtypeStruct((Q * pp, HD * C), jnp.float32),
        mesh=mesh,
        compiler_params=pltpu.CompilerParams(
            use_tc_tiling_on_sc=False, needs_layout_passes=False),
        scratch_types=[
            pltpu.VMEM((q_per_tile * HD * meta_row,), jnp.int32),
            pltpu.VMEM((q_per_tile * HD, LANES), jnp.float32),
            pltpu.VMEM((n_idx,), jnp.int32),
            pltpu.VMEM((n_idx,), jnp.int32),
            pltpu.VMEM((128,), jnp.int32),
            pltpu.VMEM((128,), jnp.int32),
            pltpu.VMEM((n_idx, LANES), jnp.float32),
            pltpu.VMEM((pp, HD * C), jnp.float32),
            pltpu.SemaphoreType.DMA,
        ],
    )
    def sc_call(table_hbm, meta_hbm, w_hbm, patk_hbm, pato_hbm, out_hbm,
                meta_v, w_v, patk_v, pato_v, idx0_v, idx1_v, rows_v,
                out_v, sem):
        wid = lax.axis_index("s") * NUM_CORES + lax.axis_index("c")
        pair0 = wid * (q_per_tile * HD)

        pltpu.sync_copy(
            meta_hbm.at[pl.ds(pair0 * meta_row, q_per_tile * HD * meta_row)],
            meta_v)
        pltpu.sync_copy(w_hbm.at[pl.ds(pair0, q_per_tile * HD)], w_v)
        pltpu.sync_copy(patk_hbm, patk_v)
        pltpu.sync_copy(pato_hbm, pato_v)

        def hd_step(hd, q_loc):
            row = q_loc * HD + hd
            mbase = jnp.full((LANES,), row * meta_row, jnp.int32)
            hd_off = hd * (T * H * W)
            for n in range(n_idx // LANES):
                kv = patk_v[pl.ds(n * LANES, LANES)]
                ov = pato_v[pl.ds(n * LANES, LANES)]
                tg = plsc.load_gather(meta_v, [kv + mbase])
                ig = plsc.load_gather(meta_v, [kv + mbase + LANES])
                jg = plsc.load_gather(meta_v, [kv + mbase + 2 * LANES])
                iv = (tg * H + ig) * W + jg + hd_off + ov
                if n < 8:
                    idx0_v[pl.ds(n * LANES, LANES)] = iv
                else:
                    idx1_v[pl.ds((n - 8) * LANES, LANES)] = iv
            c1 = pltpu.async_copy(
                table_hbm.at[idx0_v], rows_v.at[pl.ds(0, 128)], sem)
            c2 = pltpu.async_copy(
                table_hbm.at[idx1_v], rows_v.at[pl.ds(128, 128)], sem)
            c1.wait()
            c2.wait()
            accs = [jnp.zeros((LANES,), jnp.float32) for _ in range(pp)]
            wv = w_v[row, :]
            for k in range(K_TOP):
                ws = jnp.full((LANES,), wv[k], jnp.float32)
                for m in range(pp):
                    accs[m] = accs[m] + ws * rows_v[k * pp + m, :]
            for m in range(pp):
                out_v[m, pl.ds(hd * C, C)] = accs[m]
            return q_loc

        def q_step(p, carry):
            lax.fori_loop(0, HD, hd_step, p)
            q_glob = wid * q_per_tile + p
            pltpu.sync_copy(out_v, out_hbm.at[pl.ds(q_glob * pp, pp)])
            return carry

        lax.fori_loop(0, q_per_tile, q_step, 0)

    return sc_call


def kernel(vid, dists, inds):
    B, HD, T, C, H, W = vid.shape
    Q = dists.shape[2]
    pp = PS * PS

    table = jnp.transpose(vid[0], (0, 1, 3, 4, 2)).reshape(HD * T * H * W, C)

    w = dists[0, :, :, :K_TOP]
    tij = inds[0, :, :, :K_TOP, :]
    w = jnp.transpose(w, (1, 0, 2))
    tij = jnp.transpose(tij, (1, 0, 2, 3))
    w_pad = jnp.pad(w, ((0, 0), (0, 0), (0, LANES - K_TOP)))
    w_pad = w_pad.reshape(Q * HD, LANES)
    tij_pad = jnp.pad(tij, ((0, 0), (0, 0), (0, LANES - K_TOP), (0, 0)))
    meta = jnp.transpose(tij_pad, (0, 1, 3, 2)).reshape(Q * HD * 3 * LANES)

    n = np.arange(256)
    valid = n < K_TOP * pp
    k_pat = np.where(valid, n // pp, 0).astype(np.int32)
    m = np.where(valid, n % pp, 0)
    off_pat = ((m // PS) * W + (m % PS)).astype(np.int32)

    sc_call = _make_sc_call(HD, T, C, H, W, Q)
    out = sc_call(table, meta, w_pad,
                  jnp.asarray(k_pat), jnp.asarray(off_pat))
    return out.reshape(Q * pp, 1, HD * C)
